# baseline (speedup 1.0000x reference)
import functools

import jax
import jax.numpy as jnp
import numpy as np
from jax import lax
from jax.experimental import pallas as pl
from jax.experimental.pallas import tpu as pltpu
from jax.experimental.pallas import tpu_sc as plsc

_B = 4096
_S = 200
_C = 100000
_D = 128
_PAD = 208
_CHUNK = 104
_LANES = 16

_info = plsc.get_sparse_core_info()
_NC = _info.num_cores
_NS = _info.num_subcores
_NW = _NC * _NS
_RPW = _B // _NW


def _neg_idx():
    nkey = jax.random.key(12345)
    neg = jax.random.randint(nkey, (_B * _S,), 1, _C)
    return neg.astype(jnp.int32).reshape(_B, _S)


def _sc_scores(W, x, idx):
    mesh = plsc.VectorSubcoreMesh(core_axis_name="c", subcore_axis_name="s")

    @functools.partial(
        pl.kernel,
        out_type=jax.ShapeDtypeStruct((_B, _PAD, _LANES), jnp.float32),
        mesh=mesh,
        compiler_params=pltpu.CompilerParams(needs_layout_passes=False),
        scratch_types=[
            pltpu.VMEM((2, _D), jnp.float32),
            pltpu.VMEM((2, 2, _CHUNK), jnp.int32),
            pltpu.VMEM((2, _PAD, _D), jnp.float32),
            pltpu.VMEM((2, _PAD, _LANES), jnp.float32),
            pltpu.SemaphoreType.DMA,
            pltpu.SemaphoreType.DMA,
            pltpu.SemaphoreType.DMA,
            pltpu.SemaphoreType.DMA,
            pltpu.SemaphoreType.DMA,
            pltpu.SemaphoreType.DMA,
        ],
    )
    def k(W_hbm, x_hbm, idx_hbm, out_hbm, x_v, idx_v, rows_v, sc_v,
          g0, g1, o0, o1, i0, i1):
        wid = lax.axis_index("s") * _NC + lax.axis_index("c")
        base = wid * _RPW
        gsem = [g0, g1]
        osem = [o0, o1]
        isem = [i0, i1]

        def issue_row(r, p):
            pltpu.async_copy(idx_hbm.at[base + r], idx_v.at[p], isem[p])
            pltpu.async_copy(x_hbm.at[base + r], x_v.at[p], isem[p])

        def wait_row(r, p):
            pltpu.make_async_copy(idx_hbm.at[base + r], idx_v.at[p], isem[p]).wait()
            pltpu.make_async_copy(x_hbm.at[base + r], x_v.at[p], isem[p]).wait()

        def issue(p):
            for c in range(2):
                pltpu.async_copy(
                    W_hbm.at[idx_v.at[p, c]],
                    rows_v.at[p, pl.ds(c * _CHUNK, _CHUNK)],
                    gsem[p],
                )

        def wait_gather(p):
            for c in range(2):
                pltpu.make_async_copy(
                    W_hbm.at[idx_v.at[p, c]],
                    rows_v.at[p, pl.ds(c * _CHUNK, _CHUNK)],
                    gsem[p],
                ).wait()

        def compute(p, r):
            xs = [x_v[p, pl.ds(16 * k, 16)] for k in range(8)]

            def group(t, carry):
                for q in range(16):
                    j = t * 16 + q
                    a0 = xs[0] * rows_v[p, j, pl.ds(0, 16)]
                    a1 = xs[1] * rows_v[p, j, pl.ds(16, 16)]
                    for kk in range(2, 8, 2):
                        a0 = a0 + xs[kk] * rows_v[p, j, pl.ds(16 * kk, 16)]
                        a1 = a1 + xs[kk + 1] * rows_v[p, j, pl.ds(16 * kk + 16, 16)]
                    sc_v[p, j, :] = a0 + a1
                return carry

            lax.fori_loop(0, _PAD // 16, group, 0)

        issue_row(0, 0)
        issue_row(1, 1)
        wait_row(0, 0)
        issue(0)

        def body(g, carry):
            for p in range(2):
                r = 2 * g + p
                wait_gather(p)

                @pl.when(r < _RPW - 1)
                def _():
                    wait_row(r + 1, 1 - p)
                    issue(1 - p)

                @pl.when(r >= 2)
                def _():
                    pltpu.make_async_copy(
                        sc_v.at[p], out_hbm.at[base + r - 2], osem[p]
                    ).wait()

                compute(p, r)
                pltpu.async_copy(sc_v.at[p], out_hbm.at[base + r], osem[p])

                @pl.when(r < _RPW - 2)
                def _():
                    issue_row(r + 2, p)
            return carry

        lax.fori_loop(0, _RPW // 2, body, 0)
        for p in range(2):
            pltpu.make_async_copy(
                sc_v.at[p], out_hbm.at[base + _RPW - 2 + p], osem[p]
            ).wait()

    return k(W, x, idx)


_TC_BLK = 256


def _tc_loss(partials_flat):
    def body(s_ref, r_ref, o_ref, acc_ref):
        i = pl.program_id(0)
        x = s_ref[...]
        s = jax.lax.dot_general(
            x, r_ref[...], (((1,), (0,)), ((), ())),
            preferred_element_type=jnp.float32,
        )
        col = lax.broadcasted_iota(jnp.int32, (_TC_BLK, _PAD), 1)

        def logsig(z):
            return jnp.minimum(z, 0.0) - jnp.log1p(jnp.exp(-jnp.abs(z)))

        pos = jnp.where(col == 0, logsig(s), 0.0)
        neg = jnp.where((col >= 1) & (col <= _S), logsig(-s), 0.0)
        part = jnp.sum(pos + neg)

        @pl.when(i == 0)
        def _():
            acc_ref[0] = 0.0

        acc_ref[0] += part

        @pl.when(i == pl.num_programs(0) - 1)
        def _():
            o_ref[0, 0] = -acc_ref[0] / _B

    return pl.pallas_call(
        body,
        grid=(_B // _TC_BLK,),
        in_specs=[
            pl.BlockSpec((_TC_BLK, _PAD * _LANES), lambda i: (i, 0)),
            pl.BlockSpec((_PAD * _LANES, _PAD), lambda i: (0, 0)),
        ],
        out_shape=jax.ShapeDtypeStruct((1, 1), jnp.float32),
        out_specs=pl.BlockSpec((1, 1), lambda i: (0, 0), memory_space=pltpu.SMEM),
        scratch_shapes=[pltpu.SMEM((1,), jnp.float32)],
    )(partials_flat, _reduce_matrix())


def _reduce_matrix():
    group = lax.broadcasted_iota(jnp.int32, (_PAD * _LANES, _PAD), 0) // _LANES
    col = lax.broadcasted_iota(jnp.int32, (_PAD * _LANES, _PAD), 1)
    return (group == col).astype(jnp.float32)


def kernel(inputs, targets, W):
    tgt = targets.astype(jnp.int32)
    npad = _PAD - 1 - _S
    pad = (lax.broadcasted_iota(jnp.int32, (_B, npad), 0) * npad
           + lax.broadcasted_iota(jnp.int32, (_B, npad), 1)) % (_C - 1) + 1
    idx = jnp.concatenate([tgt, _neg_idx(), pad], axis=1)
    idx = idx.reshape(_B, 2, _CHUNK)
    partials = _sc_scores(W, inputs.astype(jnp.float32), idx)
    return _tc_loss(partials.reshape(_B, _PAD * _LANES))[0, 0]
